# baseline (speedup 1.0000x reference)
import functools

import jax
import jax.numpy as jnp
from jax import lax
from jax.experimental import pallas as pl
from jax.experimental.pallas import tpu as pltpu
from jax.experimental.pallas import tpu_sc as plsc

NW = 32
NTILES = 16
CHUNK = 128
D = 128
NBUF = 2
NSTAGE = 2


def _round_up(v, m):
    return (v + m - 1) // m * m


@functools.partial(jax.jit, static_argnames=("np_rows", "nchunks"))
def _sc_edge_scatter(y, srcs, dsts, zeros, *, np_rows, nchunks):
    rows_per_tile = np_rows // NTILES
    nhalf = nchunks // NSTAGE
    nblocks = nhalf // NBUF
    mesh = plsc.VectorSubcoreMesh(core_axis_name="c", subcore_axis_name="s")

    @functools.partial(
        pl.kernel,
        mesh=mesh,
        out_type=jax.ShapeDtypeStruct((2, np_rows, D), jnp.float32),
        scratch_types=[
            pltpu.VMEM((nhalf, CHUNK), jnp.int32),
            pltpu.VMEM((nhalf, CHUNK), jnp.int32),
        ] + [pltpu.VMEM((CHUNK, D), jnp.float32)] * NBUF + [
            pltpu.VMEM_SHARED((np_rows, D), jnp.float32),
        ] + [pltpu.SemaphoreType.DMA] * NBUF,
    )
    def k(y_hbm, srcs_hbm, dsts_hbm, zeros_hbm, out_hbm,
          src_v, dst_v, *rest):
        rows = rest[:NBUF]
        accum = rest[NBUF]
        sems = rest[NBUF + 1:]
        c = lax.axis_index("c")
        s = lax.axis_index("s")
        wid = s * 2 + c
        row0 = s * rows_per_tile
        pltpu.sync_copy(zeros_hbm.at[pl.ds(row0, rows_per_tile)],
                        accum.at[pl.ds(row0, rows_per_tile)])
        plsc.subcore_barrier()

        for h in range(NSTAGE):
            pltpu.sync_copy(srcs_hbm.at[wid, pl.ds(h * nhalf, nhalf)], src_v)
            pltpu.sync_copy(dsts_hbm.at[wid, pl.ds(h * nhalf, nhalf)], dst_v)
            for b in range(NBUF):
                pltpu.async_copy(y_hbm.at[src_v.at[b]], rows[b], sems[b])

            def body(i, carry):
                j = i * NBUF
                for b in range(NBUF):
                    pltpu.make_async_copy(
                        y_hbm.at[src_v.at[j + b]], rows[b], sems[b]).wait()
                    pltpu.sync_copy(rows[b], accum.at[dst_v.at[j + b]],
                                    add=True)
                    pltpu.async_copy(
                        y_hbm.at[src_v.at[j + b + NBUF]], rows[b], sems[b])
                return carry

            lax.fori_loop(0, nblocks - 1, body, 0)
            jlast = (nblocks - 1) * NBUF
            for b in range(NBUF):
                pltpu.make_async_copy(
                    y_hbm.at[src_v.at[jlast + b]], rows[b], sems[b]).wait()
                pltpu.sync_copy(rows[b], accum.at[dst_v.at[jlast + b]],
                                add=True)
        plsc.subcore_barrier()
        pltpu.sync_copy(accum.at[pl.ds(row0, rows_per_tile)],
                        out_hbm.at[c, pl.ds(row0, rows_per_tile)])

    return k(y, srcs, dsts, zeros)


@functools.partial(jax.jit, static_argnames=("np_rows", "nchunks"))
def _sc_degree(dsts, ones, zeros, *, np_rows, nchunks):
    rows_per_tile = np_rows // NTILES
    mesh = plsc.VectorSubcoreMesh(core_axis_name="c", subcore_axis_name="s")

    @functools.partial(
        pl.kernel,
        mesh=mesh,
        out_type=jax.ShapeDtypeStruct((2, np_rows, D), jnp.float32),
        scratch_types=[
            pltpu.VMEM((nchunks, CHUNK), jnp.int32),
            pltpu.VMEM((CHUNK, D), jnp.float32),
            pltpu.VMEM_SHARED((np_rows, D), jnp.float32),
        ],
    )
    def k(dsts_hbm, ones_hbm, zeros_hbm, out_hbm, dst_v, ones_v, accum):
        c = lax.axis_index("c")
        s = lax.axis_index("s")
        wid = s * 2 + c
        row0 = s * rows_per_tile
        pltpu.sync_copy(zeros_hbm.at[pl.ds(row0, rows_per_tile)],
                        accum.at[pl.ds(row0, rows_per_tile)])
        pltpu.sync_copy(dsts_hbm.at[wid], dst_v)
        pltpu.sync_copy(ones_hbm, ones_v)
        plsc.subcore_barrier()

        def body(j, carry):
            pltpu.sync_copy(ones_v, accum.at[dst_v.at[j]], add=True)
            return carry

        lax.fori_loop(0, nchunks, body, 0)
        plsc.subcore_barrier()
        pltpu.sync_copy(accum.at[pl.ds(row0, rows_per_tile)],
                        out_hbm.at[c, pl.ds(row0, rows_per_tile)])

    return k(dsts, ones, zeros)


def _tc_first(x, w1, degp, *, n, np_rows):
    def body(x_ref, w_ref, degp_ref, dinv_ref, y_ref):
        deg = degp_ref[0] + degp_ref[1] + 1.0
        dinv = lax.rsqrt(deg)
        dinv_ref[...] = dinv
        y = jnp.dot(x_ref[...], w_ref[...],
                    preferred_element_type=jnp.float32)
        y_ref[0:n, :] = y * dinv
        y_ref[n:np_rows, :] = jnp.zeros(
            (np_rows - n, y_ref.shape[1]), jnp.float32)

    return pl.pallas_call(
        body,
        out_shape=(
            jax.ShapeDtypeStruct((n, 1), jnp.float32),
            jax.ShapeDtypeStruct((np_rows, D), jnp.float32),
        ),
    )(x, w1, degp)


def _tc_mid(p, yprev, dinv, b, w, *, n, np_rows):
    def body(p_ref, y_ref, dinv_ref, b_ref, w_ref, out_ref):
        s = p_ref[0] + p_ref[1] + y_ref[...]
        h = dinv_ref[...] * s + b_ref[...]
        y = jnp.dot(h, w_ref[...], preferred_element_type=jnp.float32)
        out_ref[0:n, :] = y * dinv_ref[...]
        out_ref[n:np_rows, :] = jnp.zeros(
            (np_rows - n, out_ref.shape[1]), jnp.float32)

    return pl.pallas_call(
        body,
        out_shape=jax.ShapeDtypeStruct((np_rows, D), jnp.float32),
    )(p, yprev, dinv, b, w)


def _tc_last(p, yprev, dinv, b, *, n):
    def body(p_ref, y_ref, dinv_ref, b_ref, out_ref):
        s = p_ref[0] + p_ref[1] + y_ref[...]
        out_ref[...] = dinv_ref[...] * s + b_ref[...]

    return pl.pallas_call(
        body,
        out_shape=jax.ShapeDtypeStruct((n, 1), jnp.float32),
    )(p, yprev, dinv, b)


def kernel(x, edge_index, W1, b1, W2, b2, W3, b3):
    n = x.shape[0]
    e = edge_index.shape[1]
    np_rows = _round_up(n + 1, 128)
    nchunks = _round_up(-(-e // (NW * CHUNK)), 2 * NSTAGE * NBUF)
    epad = NW * nchunks * CHUNK

    src = jnp.asarray(edge_index[0], jnp.int32)
    dst = jnp.asarray(edge_index[1], jnp.int32)
    fill = jnp.full((epad - e,), n, jnp.int32)
    srcs = jnp.concatenate([src, fill]).reshape(NW, nchunks, CHUNK)
    dsts = jnp.concatenate([dst, fill]).reshape(NW, nchunks, CHUNK)

    d1 = W1.shape[1]
    d3 = W3.shape[1]
    w1p = jnp.zeros((x.shape[1], D), jnp.float32).at[:, 0:d1].set(W1)
    w2p = jnp.zeros((d1, D), jnp.float32).at[:, 0:d1].set(W2)
    w3p = jnp.zeros((d1, D), jnp.float32).at[:, 0:d3].set(W3)
    b1p = jnp.zeros((1, d1), jnp.float32).at[0].set(b1)
    b2p = jnp.zeros((1, d1), jnp.float32).at[0].set(b2)

    zeros_t = jnp.zeros((np_rows, D), jnp.float32)
    ones_c = jnp.ones((CHUNK, D), jnp.float32)

    degp = _sc_degree(dsts, ones_c, zeros_t, np_rows=np_rows, nchunks=nchunks)
    dinv, y1 = _tc_first(x, w1p, degp[:, 0:n, 0:1], n=n, np_rows=np_rows)

    p1 = _sc_edge_scatter(y1, srcs, dsts, zeros_t,
                          np_rows=np_rows, nchunks=nchunks)
    y2 = _tc_mid(p1[:, 0:n, 0:d1], y1[0:n, 0:d1], dinv, b1p, w2p,
                 n=n, np_rows=np_rows)

    p2 = _sc_edge_scatter(y2, srcs, dsts, zeros_t,
                          np_rows=np_rows, nchunks=nchunks)
    y3 = _tc_mid(p2[:, 0:n, 0:d1], y2[0:n, 0:d1], dinv, b2p, w3p,
                 n=n, np_rows=np_rows)

    p3 = _sc_edge_scatter(y3, srcs, dsts, zeros_t,
                          np_rows=np_rows, nchunks=nchunks)
    return _tc_last(p3[:, 0:n, 0:d3], y3[0:n, 0:d3], dinv,
                    b3.reshape(1, -1), n=n)


# speedup vs baseline: 1.2206x; 1.2206x over previous
import functools

import jax
import jax.numpy as jnp
from jax import lax
from jax.experimental import pallas as pl
from jax.experimental.pallas import tpu as pltpu
from jax.experimental.pallas import tpu_sc as plsc

NW = 32
NTILES = 16
CHUNK = 128
TW = 128


def _round_up(v, m):
    return (v + m - 1) // m * m


@functools.partial(jax.jit, static_argnames=("np_rows", "nchunks"))
def _sc_edge_scatter(y, srcs, dsts, zeros, *, np_rows, nchunks):
    rows_per_tile = np_rows // NTILES
    mesh = plsc.VectorSubcoreMesh(core_axis_name="c", subcore_axis_name="s")

    @functools.partial(
        pl.kernel,
        mesh=mesh,
        out_type=jax.ShapeDtypeStruct((2, np_rows, TW), jnp.float32),
        scratch_types=[
            pltpu.VMEM((nchunks, CHUNK), jnp.int32),
            pltpu.VMEM((nchunks, CHUNK), jnp.int32),
            pltpu.VMEM((CHUNK, TW), jnp.float32),
            pltpu.VMEM_SHARED((np_rows, TW), jnp.float32),
        ],
    )
    def k(y_hbm, srcs_hbm, dsts_hbm, zeros_hbm, out_hbm,
          src_v, dst_v, rows_v, accum):
        c = lax.axis_index("c")
        s = lax.axis_index("s")
        wid = s * 2 + c
        row0 = s * rows_per_tile
        pltpu.sync_copy(zeros_hbm.at[pl.ds(row0, rows_per_tile)],
                        accum.at[pl.ds(row0, rows_per_tile)])
        pltpu.sync_copy(srcs_hbm.at[wid], src_v)
        pltpu.sync_copy(dsts_hbm.at[wid], dst_v)
        plsc.subcore_barrier()

        def body(j, carry):
            pltpu.sync_copy(y_hbm.at[src_v.at[j]], rows_v)
            pltpu.sync_copy(rows_v, accum.at[dst_v.at[j]], add=True)
            return carry

        lax.fori_loop(0, nchunks, body, 0)
        plsc.subcore_barrier()
        pltpu.sync_copy(accum.at[pl.ds(row0, rows_per_tile)],
                        out_hbm.at[c, pl.ds(row0, rows_per_tile)])

    return k(y, srcs, dsts, zeros)


@functools.partial(jax.jit, static_argnames=("np_rows", "nchunks"))
def _sc_degree(dsts, ones, zeros, *, np_rows, nchunks):
    rows_per_tile = np_rows // NTILES
    mesh = plsc.VectorSubcoreMesh(core_axis_name="c", subcore_axis_name="s")

    @functools.partial(
        pl.kernel,
        mesh=mesh,
        out_type=jax.ShapeDtypeStruct((2, np_rows, TW), jnp.float32),
        scratch_types=[
            pltpu.VMEM((nchunks, CHUNK), jnp.int32),
            pltpu.VMEM((CHUNK, TW), jnp.float32),
            pltpu.VMEM_SHARED((np_rows, TW), jnp.float32),
        ],
    )
    def k(dsts_hbm, ones_hbm, zeros_hbm, out_hbm, dst_v, ones_v, accum):
        c = lax.axis_index("c")
        s = lax.axis_index("s")
        wid = s * 2 + c
        row0 = s * rows_per_tile
        pltpu.sync_copy(zeros_hbm.at[pl.ds(row0, rows_per_tile)],
                        accum.at[pl.ds(row0, rows_per_tile)])
        pltpu.sync_copy(dsts_hbm.at[wid], dst_v)
        pltpu.sync_copy(ones_hbm, ones_v)
        plsc.subcore_barrier()

        def body(j, carry):
            pltpu.sync_copy(ones_v, accum.at[dst_v.at[j]], add=True)
            return carry

        lax.fori_loop(0, nchunks, body, 0)
        plsc.subcore_barrier()
        pltpu.sync_copy(accum.at[pl.ds(row0, rows_per_tile)],
                        out_hbm.at[c, pl.ds(row0, rows_per_tile)])

    return k(dsts, ones, zeros)


def _tc_first(x, w1, degp, *, n, np_rows):
    def body(x_ref, w_ref, degp_ref, dinv_ref, y_ref):
        deg = degp_ref[0] + degp_ref[1] + 1.0
        dinv = lax.rsqrt(deg)
        dinv_ref[...] = dinv
        y = jnp.dot(x_ref[...], w_ref[...],
                    preferred_element_type=jnp.float32)
        y_ref[0:n, :] = y * dinv
        y_ref[n:np_rows, :] = jnp.zeros(
            (np_rows - n, y_ref.shape[1]), jnp.float32)

    return pl.pallas_call(
        body,
        out_shape=(
            jax.ShapeDtypeStruct((n, 1), jnp.float32),
            jax.ShapeDtypeStruct((np_rows, TW), jnp.float32),
        ),
    )(x, w1, degp)


def _tc_mid(p, yprev, dinv, b, w, *, n, np_rows):
    def body(p_ref, y_ref, dinv_ref, b_ref, w_ref, out_ref):
        s = p_ref[0] + p_ref[1] + y_ref[...]
        h = dinv_ref[...] * s + b_ref[...]
        y = jnp.dot(h, w_ref[...], preferred_element_type=jnp.float32)
        out_ref[0:n, :] = y * dinv_ref[...]
        out_ref[n:np_rows, :] = jnp.zeros(
            (np_rows - n, out_ref.shape[1]), jnp.float32)

    return pl.pallas_call(
        body,
        out_shape=jax.ShapeDtypeStruct((np_rows, TW), jnp.float32),
    )(p, yprev, dinv, b, w)


def _tc_last(p, yprev, dinv, b, *, n):
    def body(p_ref, y_ref, dinv_ref, b_ref, out_ref):
        s = p_ref[0] + p_ref[1] + y_ref[...]
        out_ref[...] = dinv_ref[...] * s + b_ref[...]

    return pl.pallas_call(
        body,
        out_shape=jax.ShapeDtypeStruct((n, 1), jnp.float32),
    )(p, yprev, dinv, b)


def kernel(x, edge_index, W1, b1, W2, b2, W3, b3):
    n = x.shape[0]
    e = edge_index.shape[1]
    np_rows = _round_up(n + 1, 128)
    nchunks = -(-e // (NW * CHUNK))
    epad = NW * nchunks * CHUNK

    src = jnp.asarray(edge_index[0], jnp.int32)
    dst = jnp.asarray(edge_index[1], jnp.int32)
    fill = jnp.full((epad - e,), n, jnp.int32)
    srcs = jnp.concatenate([src, fill]).reshape(NW, nchunks, CHUNK)
    dsts = jnp.concatenate([dst, fill]).reshape(NW, nchunks, CHUNK)

    d1 = W1.shape[1]
    d3 = W3.shape[1]
    w1p = jnp.zeros((x.shape[1], TW), jnp.float32).at[:, 0:d1].set(W1)
    w2p = jnp.zeros((TW, TW), jnp.float32).at[0:d1, 0:d1].set(W2)
    w3p = jnp.zeros((TW, TW), jnp.float32).at[0:d1, 0:d3].set(W3)
    b1p = jnp.zeros((1, TW), jnp.float32).at[0, 0:d1].set(b1)
    b2p = jnp.zeros((1, TW), jnp.float32).at[0, 0:d1].set(b2)

    zeros_t = jnp.zeros((np_rows, TW), jnp.float32)
    ones_c = jnp.ones((CHUNK, TW), jnp.float32)

    degp = _sc_degree(dsts, ones_c, zeros_t, np_rows=np_rows, nchunks=nchunks)
    dinv, y1 = _tc_first(x, w1p, degp[:, 0:n, 0:1], n=n, np_rows=np_rows)

    p1 = _sc_edge_scatter(y1, srcs, dsts, zeros_t,
                          np_rows=np_rows, nchunks=nchunks)
    y2 = _tc_mid(p1[:, 0:n, :], y1[0:n, :], dinv, b1p, w2p,
                 n=n, np_rows=np_rows)

    p2 = _sc_edge_scatter(y2, srcs, dsts, zeros_t,
                          np_rows=np_rows, nchunks=nchunks)
    y3 = _tc_mid(p2[:, 0:n, :], y2[0:n, :], dinv, b2p, w3p,
                 n=n, np_rows=np_rows)

    p3 = _sc_edge_scatter(y3, srcs, dsts, zeros_t,
                          np_rows=np_rows, nchunks=nchunks)
    return _tc_last(p3[:, 0:n, 0:d3], y3[0:n, 0:d3], dinv,
                    b3.reshape(1, -1), n=n)


# speedup vs baseline: 1.4438x; 1.1828x over previous
import functools

import jax
import jax.numpy as jnp
from jax import lax
from jax.experimental import pallas as pl
from jax.experimental.pallas import tpu as pltpu
from jax.experimental.pallas import tpu_sc as plsc

NW = 32
NTILES = 16
CHUNK = 128
TW = 128


def _round_up(v, m):
    return (v + m - 1) // m * m


@functools.partial(jax.jit, static_argnames=("np_rows", "nchunks"))
def _sc_edge_scatter(y, srcs, dsts, zeros, *, np_rows, nchunks):
    rows_per_tile = np_rows // NTILES
    mesh = plsc.VectorSubcoreMesh(core_axis_name="c", subcore_axis_name="s")

    @functools.partial(
        pl.kernel,
        mesh=mesh,
        out_type=jax.ShapeDtypeStruct((2, np_rows, TW), jnp.float32),
        scratch_types=[
            pltpu.VMEM((nchunks, CHUNK), jnp.int32),
            pltpu.VMEM((nchunks, CHUNK), jnp.int32),
            pltpu.VMEM((CHUNK, TW), jnp.float32),
            pltpu.VMEM_SHARED((np_rows, TW), jnp.float32),
        ],
    )
    def k(y_hbm, srcs_hbm, dsts_hbm, zeros_hbm, out_hbm,
          src_v, dst_v, rows_v, accum):
        c = lax.axis_index("c")
        s = lax.axis_index("s")
        wid = s * 2 + c
        row0 = s * rows_per_tile
        pltpu.sync_copy(zeros_hbm.at[pl.ds(row0, rows_per_tile)],
                        accum.at[pl.ds(row0, rows_per_tile)])
        pltpu.sync_copy(srcs_hbm.at[wid], src_v)
        pltpu.sync_copy(dsts_hbm.at[wid], dst_v)
        plsc.subcore_barrier()

        def body(j, carry):
            pltpu.sync_copy(y_hbm.at[src_v.at[j]], rows_v)
            pltpu.sync_copy(rows_v, accum.at[dst_v.at[j]], add=True)
            return carry

        lax.fori_loop(0, nchunks, body, 0)
        plsc.subcore_barrier()
        pltpu.sync_copy(accum.at[pl.ds(row0, rows_per_tile)],
                        out_hbm.at[c, pl.ds(row0, rows_per_tile)])

    return k(y, srcs, dsts, zeros)


@functools.partial(jax.jit, static_argnames=("np_rows", "nchunks"))
def _sc_degree(dsts, ones, zeros, *, np_rows, nchunks):
    rows_per_tile = np_rows // NTILES
    mesh = plsc.VectorSubcoreMesh(core_axis_name="c", subcore_axis_name="s")

    @functools.partial(
        pl.kernel,
        mesh=mesh,
        out_type=jax.ShapeDtypeStruct((2, np_rows, TW), jnp.float32),
        scratch_types=[
            pltpu.VMEM((nchunks, CHUNK), jnp.int32),
            pltpu.VMEM((CHUNK, TW), jnp.float32),
            pltpu.VMEM_SHARED((np_rows, TW), jnp.float32),
        ],
    )
    def k(dsts_hbm, ones_hbm, zeros_hbm, out_hbm, dst_v, ones_v, accum):
        c = lax.axis_index("c")
        s = lax.axis_index("s")
        wid = s * 2 + c
        row0 = s * rows_per_tile
        pltpu.sync_copy(zeros_hbm.at[pl.ds(row0, rows_per_tile)],
                        accum.at[pl.ds(row0, rows_per_tile)])
        pltpu.sync_copy(dsts_hbm.at[wid], dst_v)
        pltpu.sync_copy(ones_hbm, ones_v)
        plsc.subcore_barrier()

        def body(j, carry):
            pltpu.sync_copy(ones_v, accum.at[dst_v.at[j]], add=True)
            return carry

        lax.fori_loop(0, nchunks, body, 0)
        plsc.subcore_barrier()
        pltpu.sync_copy(accum.at[pl.ds(row0, rows_per_tile)],
                        out_hbm.at[c, pl.ds(row0, rows_per_tile)])

    return k(dsts, ones, zeros)


def _tc_first(x, w1, degp, *, n, np_rows):
    def body(x_ref, w_ref, degp_ref, dinv_ref, y_ref):
        deg = degp_ref[0, 0:n, 0:1] + degp_ref[1, 0:n, 0:1] + 1.0
        dinv = lax.rsqrt(deg)
        dinv_ref[...] = dinv
        y = jnp.dot(x_ref[...], w_ref[...],
                    preferred_element_type=jnp.float32)
        y_ref[0:n, :] = y * dinv
        y_ref[n:np_rows, :] = jnp.zeros(
            (np_rows - n, y_ref.shape[1]), jnp.float32)

    return pl.pallas_call(
        body,
        out_shape=(
            jax.ShapeDtypeStruct((n, 1), jnp.float32),
            jax.ShapeDtypeStruct((np_rows, TW), jnp.float32),
        ),
    )(x, w1, degp)


def _tc_mid(p, yprev, dinv, b, w, *, n, np_rows):
    def body(p_ref, y_ref, dinv_ref, b_ref, w_ref, out_ref):
        s = p_ref[0, 0:n, :] + p_ref[1, 0:n, :] + y_ref[0:n, :]
        h = dinv_ref[...] * s + b_ref[...]
        y = jnp.dot(h, w_ref[...], preferred_element_type=jnp.float32)
        out_ref[0:n, :] = y * dinv_ref[...]
        out_ref[n:np_rows, :] = jnp.zeros(
            (np_rows - n, out_ref.shape[1]), jnp.float32)

    return pl.pallas_call(
        body,
        out_shape=jax.ShapeDtypeStruct((np_rows, TW), jnp.float32),
    )(p, yprev, dinv, b, w)


def _tc_last(p, yprev, dinv, b, *, n):
    def body(p_ref, y_ref, dinv_ref, b_ref, out_ref):
        s = p_ref[0, 0:n, 0:1] + p_ref[1, 0:n, 0:1] + y_ref[0:n, 0:1]
        out_ref[...] = dinv_ref[...] * s + b_ref[...]

    return pl.pallas_call(
        body,
        out_shape=jax.ShapeDtypeStruct((n, 1), jnp.float32),
    )(p, yprev, dinv, b)


def kernel(x, edge_index, W1, b1, W2, b2, W3, b3):
    n = x.shape[0]
    e = edge_index.shape[1]
    np_rows = _round_up(n + 1, 128)
    nchunks = -(-e // (NW * CHUNK))
    epad = NW * nchunks * CHUNK

    src = jnp.asarray(edge_index[0], jnp.int32)
    dst = jnp.asarray(edge_index[1], jnp.int32)
    fill = jnp.full((epad - e,), n, jnp.int32)
    srcs = jnp.concatenate([src, fill]).reshape(NW, nchunks, CHUNK)
    dsts = jnp.concatenate([dst, fill]).reshape(NW, nchunks, CHUNK)

    d1 = W1.shape[1]
    d3 = W3.shape[1]
    w1p = jnp.zeros((x.shape[1], TW), jnp.float32).at[:, 0:d1].set(W1)
    w2p = jnp.zeros((TW, TW), jnp.float32).at[0:d1, 0:d1].set(W2)
    w3p = jnp.zeros((TW, TW), jnp.float32).at[0:d1, 0:d3].set(W3)
    b1p = jnp.zeros((1, TW), jnp.float32).at[0, 0:d1].set(b1)
    b2p = jnp.zeros((1, TW), jnp.float32).at[0, 0:d1].set(b2)

    zeros_t = jnp.zeros((np_rows, TW), jnp.float32)
    ones_c = jnp.ones((CHUNK, TW), jnp.float32)

    degp = _sc_degree(dsts, ones_c, zeros_t, np_rows=np_rows, nchunks=nchunks)
    dinv, y1 = _tc_first(x, w1p, degp, n=n, np_rows=np_rows)

    p1 = _sc_edge_scatter(y1, srcs, dsts, zeros_t,
                          np_rows=np_rows, nchunks=nchunks)
    y2 = _tc_mid(p1, y1, dinv, b1p, w2p, n=n, np_rows=np_rows)

    p2 = _sc_edge_scatter(y2, srcs, dsts, zeros_t,
                          np_rows=np_rows, nchunks=nchunks)
    y3 = _tc_mid(p2, y2, dinv, b2p, w3p, n=n, np_rows=np_rows)

    p3 = _sc_edge_scatter(y3, srcs, dsts, zeros_t,
                          np_rows=np_rows, nchunks=nchunks)
    return _tc_last(p3, y3, dinv, b3.reshape(1, -1), n=n)
